# baseline (speedup 1.0000x reference)
import functools

import jax
import jax.numpy as jnp
from jax import lax
from jax.experimental import pallas as pl
from jax.experimental.pallas import tpu as pltpu
from jax.experimental.pallas import tpu_sc as plsc

B, N, E, MAX_NB, H, F_ATOM, F_BOND, DEPTH = 4, 12500, 12500, 10, 128, 89, 5, 3
NPAD = 12800
NP = B * NPAD
NEB = 13312
NPE = B * NEB
NSENT = NEB - N
NW = 32
CROWS = 16
SUB = 80
CG = CROWS * MAX_NB
RPW = NP // NW
NCH = RPW // CROWS
NEG = -1e30
TM = 512


def _bdot(a, b):
    return jnp.dot(a.astype(jnp.bfloat16), b.astype(jnp.bfloat16),
                   preferred_element_type=jnp.float32)


def _mm_init_body(x_ref, wat_ref, w2a_ref, b2_ref, af_ref, a_ref):
    af = _bdot(x_ref[0], wat_ref[...])
    af_ref[...] = af
    a_ref[...] = _bdot(af, w2a_ref[...]) + b2_ref[...]


def _mm_step_body(af_ref, nei_ref, w1a_ref, w1b_ref, b1_ref, w2a_ref, b2_ref,
                  af_out, a_out):
    x = _bdot(af_ref[...], w1a_ref[...]) + _bdot(nei_ref[...], w1b_ref[...])
    x = jnp.maximum(x + b1_ref[...], 0.0)
    af_out[...] = x
    a_out[...] = _bdot(x, w2a_ref[...]) + b2_ref[...]


def _mm_final_body(af_ref, nei_ref, w1a_ref, w1b_ref, b1_ref, out_ref):
    x = _bdot(af_ref[...], w1a_ref[...]) + _bdot(nei_ref[...], w1b_ref[...])
    out_ref[0] = jnp.maximum(x + b1_ref[...], 0.0)


def _mm_bond_body(bond_ref, w2b_ref, out_ref):
    v = _bdot(bond_ref[0], w2b_ref[...])
    row = pl.program_id(1) * TM + lax.broadcasted_iota(jnp.int32, (TM, H), 0)
    out_ref[...] = jnp.where(row < E, v, NEG)


def _full2(shape):
    n = len(shape)

    def im(*_):
        return (0,) * n

    return pl.BlockSpec(shape, im)


def _tc_init(x3, wat, w2a, b2):
    return pl.pallas_call(
        _mm_init_body,
        grid=(B, NPAD // TM),
        in_specs=[pl.BlockSpec((1, TM, F_ATOM), lambda b, j: (b, j, 0)),
                  _full2((F_ATOM, H)), _full2((H, H)), _full2((1, H))],
        out_specs=[pl.BlockSpec((TM, H), lambda b, j: (b * (NPAD // TM) + j, 0)),
                   pl.BlockSpec((TM, H), lambda b, j: (b * (NPAD // TM) + j, 0))],
        out_shape=[jax.ShapeDtypeStruct((NP, H), jnp.float32),
                   jax.ShapeDtypeStruct((NP, H), jnp.float32)],
    )(x3, wat, w2a, b2)


def _tc_bond(bond3, w2b):
    return pl.pallas_call(
        _mm_bond_body,
        grid=(B, NEB // TM),
        in_specs=[pl.BlockSpec((1, TM, F_BOND),
                               lambda b, j: (b, jnp.minimum(j, NPAD // TM - 1), 0)),
                  _full2((F_BOND, H))],
        out_specs=pl.BlockSpec((TM, H), lambda b, j: (b * (NEB // TM) + j, 0)),
        out_shape=jax.ShapeDtypeStruct((NPE, H), jnp.float32),
    )(bond3, w2b)


def _tc_step(af, nei, w1a, w1b, b1, w2a, b2):
    rows = pl.BlockSpec((TM, H), lambda i: (i, 0))
    return pl.pallas_call(
        _mm_step_body,
        grid=(NP // TM,),
        in_specs=[rows, rows, _full2((H, H)), _full2((H, H)), _full2((1, H)),
                  _full2((H, H)), _full2((1, H))],
        out_specs=[rows, rows],
        out_shape=[jax.ShapeDtypeStruct((NP, H), jnp.float32),
                   jax.ShapeDtypeStruct((NP, H), jnp.float32)],
    )(af, nei, w1a, w1b, b1, w2a, b2)


def _tc_final(af, nei, w1a, w1b, b1):
    rows = pl.BlockSpec((TM, H), lambda b, j: (b * (NPAD // TM) + j, 0))
    return pl.pallas_call(
        _mm_final_body,
        grid=(B, NPAD // TM),
        in_specs=[rows, rows, _full2((H, H)), _full2((H, H)), _full2((1, H))],
        out_specs=pl.BlockSpec((1, TM, H), lambda b, j: (b, j, 0)),
        out_shape=jax.ShapeDtypeStruct((B, N, H), jnp.float32),
    )(af, nei, w1a, w1b, b1)


def _sc_nei(A, Bv, a_idx, e_idx):
    info = plsc.get_sparse_core_info()
    nc = info.num_cores
    mesh = plsc.VectorSubcoreMesh(core_axis_name="c", subcore_axis_name="s")
    gpw = RPW * MAX_NB

    @functools.partial(
        pl.kernel,
        mesh=mesh,
        out_type=jax.ShapeDtypeStruct((NP, H), jnp.float32),
        scratch_types=[
            pltpu.VMEM((gpw,), jnp.int32),
            pltpu.VMEM((gpw,), jnp.int32),
            pltpu.VMEM((CG, H), jnp.float32),
            pltpu.VMEM((CG, H), jnp.float32),
            pltpu.VMEM((CG, H), jnp.float32),
            pltpu.VMEM((CG, H), jnp.float32),
            pltpu.VMEM((CROWS, H), jnp.float32),
            pltpu.VMEM((CROWS, H), jnp.float32),
            pltpu.SemaphoreType.DMA,
            pltpu.SemaphoreType.DMA,
        ],
    )
    def k(a_hbm, bv_hbm, ai_hbm, ei_hbm, out_hbm,
          ai_v, ei_v, ga0, gb0, ga1, gb1, oc0, oc1, sg0, sg1):
        wid = lax.axis_index("s") * nc + lax.axis_index("c")
        row_base = wid * RPW
        bufs = ((ga0, gb0, oc0, sg0), (ga1, gb1, oc1, sg1))

        pltpu.sync_copy(ai_hbm.at[pl.ds(row_base * MAX_NB, gpw)], ai_v)
        pltpu.sync_copy(ei_hbm.at[pl.ds(row_base * MAX_NB, gpw)], ei_v)

        def fire(c, b):
            ga, gb, _, sg = bufs[b]
            g0 = c * CG
            for j in range(CG // SUB):
                s = pl.ds(g0 + j * SUB, SUB)
                dsl = pl.ds(j * SUB, SUB)
                pltpu.async_copy(a_hbm.at[ai_v.at[s]], ga.at[dsl], sg)
                pltpu.async_copy(bv_hbm.at[ei_v.at[s]], gb.at[dsl], sg)

        def drain(b):
            ga, gb, _, sg = bufs[b]
            pltpu.make_async_copy(a_hbm.at[ai_v.at[pl.ds(0, CG)]], ga, sg).wait()
            pltpu.make_async_copy(bv_hbm.at[ei_v.at[pl.ds(0, CG)]], gb, sg).wait()

        def compute_store(c, b):
            ga, gb, oc, _ = bufs[b]

            def row(r, c2):
                rk = r * MAX_NB
                for h in range(H // 16):
                    sl = pl.ds(h * 16, 16)
                    acc = jnp.zeros((16,), jnp.float32)
                    for kk in range(MAX_NB):
                        v = ga[rk + kk, sl] + gb[rk + kk, sl]
                        acc = acc + jnp.maximum(v, 0.0)
                    oc[r, sl] = acc
                return c2

            lax.fori_loop(0, CROWS, row, 0)
            pltpu.sync_copy(oc, out_hbm.at[pl.ds(row_base + c * CROWS, CROWS)])

        fire(0, 0)

        def pair(i, carry):
            c0 = 2 * i
            fire(c0 + 1, 1)
            drain(0)
            compute_store(c0, 0)
            fire(c0 + 2, 0)
            drain(1)
            compute_store(c0 + 1, 1)
            return carry

        lax.fori_loop(0, NCH // 2 - 1, pair, 0)
        fire(NCH - 1, 1)
        drain(0)
        compute_store(NCH - 2, 0)
        drain(1)
        compute_store(NCH - 1, 1)

    return k(A, Bv, a_idx, e_idx)


def kernel(input_atom, input_bond, atom_graph, bond_graph, num_nbs,
           W_atom, W_U2, b_U2, W_U1, b_U1):
    w2a = W_U2[:H]
    w2b = W_U2[H:]
    w1a = W_U1[:H]
    w1b = W_U1[H:]
    b1 = b_U1.reshape(1, H)
    b2 = b_U2.reshape(1, H)

    a_core = atom_graph[..., 0] * NPAD + atom_graph[..., 1]
    valid = jnp.arange(MAX_NB, dtype=jnp.int32)[None, None, :] < num_nbs[:, :, None]
    boff_e = jnp.arange(B, dtype=jnp.int32)[:, None, None] * NEB
    spread = (jnp.arange(N * MAX_NB, dtype=jnp.int32) % NSENT).reshape(1, N, MAX_NB)
    e_core = jnp.where(valid, bond_graph[..., 0] * NEB + bond_graph[..., 1],
                       boff_e + N + spread)
    npr = NPAD - N
    boff_a = jnp.arange(B, dtype=jnp.int32)[:, None, None] * NPAD
    pidx = jnp.arange(npr * MAX_NB, dtype=jnp.int32).reshape(1, npr, MAX_NB)
    a_pad = jnp.broadcast_to(boff_a + pidx % N, (B, npr, MAX_NB))
    e_pad = jnp.broadcast_to(boff_e + N + pidx % NSENT, (B, npr, MAX_NB))
    a_flat = jnp.concatenate([a_core, a_pad], axis=1).reshape(-1)
    e_flat = jnp.concatenate([e_core, e_pad], axis=1).reshape(-1)

    bv = _tc_bond(input_bond, w2b)
    af, a_tab = _tc_init(input_atom, W_atom, w2a, b2)
    for d in range(DEPTH):
        nei = _sc_nei(a_tab, bv, a_flat, e_flat)
        if d < DEPTH - 1:
            af, a_tab = _tc_step(af, nei, w1a, w1b, b1, w2a, b2)
        else:
            out = _tc_final(af, nei, w1a, w1b, b1)
    return out


# speedup vs baseline: 1.0088x; 1.0088x over previous
import functools

import jax
import jax.numpy as jnp
from jax import lax
from jax.experimental import pallas as pl
from jax.experimental.pallas import tpu as pltpu
from jax.experimental.pallas import tpu_sc as plsc

B, N, E, MAX_NB, H, F_ATOM, F_BOND, DEPTH = 4, 12500, 12500, 10, 128, 89, 5, 3
NPAD = 12800
NP = B * NPAD
NEB = 13312
NPE = B * NEB
NSENT = NEB - N
NW = 32
CROWS = 16
SUB = 80
CG = CROWS * MAX_NB
RPW = NP // NW
NCH = RPW // CROWS
NEG = -1e30
TM = 512


def _bdot(a, b):
    return jnp.dot(a.astype(jnp.bfloat16), b.astype(jnp.bfloat16),
                   preferred_element_type=jnp.float32)


def _mm_init_body(x_ref, wat_ref, w2a_ref, b2_ref, af_ref, a_ref):
    af = _bdot(x_ref[0], wat_ref[...])
    af_ref[...] = af.astype(jnp.bfloat16)
    a_ref[...] = _bdot(af, w2a_ref[...]) + b2_ref[...]


def _mm_step_body(af_ref, nei_ref, w1a_ref, w1b_ref, b1_ref, w2a_ref, b2_ref,
                  af_out, a_out):
    x = _bdot(af_ref[...], w1a_ref[...]) + _bdot(nei_ref[...], w1b_ref[...])
    x = jnp.maximum(x + b1_ref[...], 0.0)
    af_out[...] = x.astype(jnp.bfloat16)
    a_out[...] = _bdot(x, w2a_ref[...]) + b2_ref[...]


def _mm_final_body(af_ref, nei_ref, w1a_ref, w1b_ref, b1_ref, out_ref):
    x = _bdot(af_ref[...], w1a_ref[...]) + _bdot(nei_ref[...], w1b_ref[...])
    out_ref[0] = jnp.maximum(x + b1_ref[...], 0.0)


def _mm_bond_body(bond_ref, w2b_ref, out_ref):
    v = _bdot(bond_ref[0], w2b_ref[...])
    row = pl.program_id(1) * TM + lax.broadcasted_iota(jnp.int32, (TM, H), 0)
    out_ref[...] = jnp.where(row < E, v, NEG)


def _full2(shape):
    n = len(shape)

    def im(*_):
        return (0,) * n

    return pl.BlockSpec(shape, im)


def _tc_init(x3, wat, w2a, b2):
    return pl.pallas_call(
        _mm_init_body,
        grid=(B, NPAD // TM),
        in_specs=[pl.BlockSpec((1, TM, F_ATOM), lambda b, j: (b, j, 0)),
                  _full2((F_ATOM, H)), _full2((H, H)), _full2((1, H))],
        out_specs=[pl.BlockSpec((TM, H), lambda b, j: (b * (NPAD // TM) + j, 0)),
                   pl.BlockSpec((TM, H), lambda b, j: (b * (NPAD // TM) + j, 0))],
        out_shape=[jax.ShapeDtypeStruct((NP, H), jnp.bfloat16),
                   jax.ShapeDtypeStruct((NP, H), jnp.float32)],
    )(x3, wat, w2a, b2)


def _tc_bond(bond3, w2b):
    return pl.pallas_call(
        _mm_bond_body,
        grid=(B, NEB // TM),
        in_specs=[pl.BlockSpec((1, TM, F_BOND),
                               lambda b, j: (b, jnp.minimum(j, NPAD // TM - 1), 0)),
                  _full2((F_BOND, H))],
        out_specs=pl.BlockSpec((TM, H), lambda b, j: (b * (NEB // TM) + j, 0)),
        out_shape=jax.ShapeDtypeStruct((NPE, H), jnp.float32),
    )(bond3, w2b)


def _tc_step(af, nei, w1a, w1b, b1, w2a, b2):
    rows = pl.BlockSpec((TM, H), lambda i: (i, 0))
    return pl.pallas_call(
        _mm_step_body,
        grid=(NP // TM,),
        in_specs=[rows, rows, _full2((H, H)), _full2((H, H)), _full2((1, H)),
                  _full2((H, H)), _full2((1, H))],
        out_specs=[rows, rows],
        out_shape=[jax.ShapeDtypeStruct((NP, H), jnp.bfloat16),
                   jax.ShapeDtypeStruct((NP, H), jnp.float32)],
    )(af, nei, w1a, w1b, b1, w2a, b2)


def _tc_final(af, nei, w1a, w1b, b1):
    rows = pl.BlockSpec((TM, H), lambda b, j: (b * (NPAD // TM) + j, 0))
    return pl.pallas_call(
        _mm_final_body,
        grid=(B, NPAD // TM),
        in_specs=[rows, rows, _full2((H, H)), _full2((H, H)), _full2((1, H))],
        out_specs=pl.BlockSpec((1, TM, H), lambda b, j: (b, j, 0)),
        out_shape=jax.ShapeDtypeStruct((B, N, H), jnp.float32),
    )(af, nei, w1a, w1b, b1)


def _sc_nei(A, Bv, a_idx, e_idx):
    info = plsc.get_sparse_core_info()
    nc = info.num_cores
    mesh = plsc.VectorSubcoreMesh(core_axis_name="c", subcore_axis_name="s")
    gpw = RPW * MAX_NB

    @functools.partial(
        pl.kernel,
        mesh=mesh,
        out_type=jax.ShapeDtypeStruct((NP, H), jnp.float32),
        scratch_types=[
            pltpu.VMEM((gpw,), jnp.int32),
            pltpu.VMEM((gpw,), jnp.int32),
            pltpu.VMEM((CG, H), jnp.float32),
            pltpu.VMEM((CG, H), jnp.float32),
            pltpu.VMEM((CG, H), jnp.float32),
            pltpu.VMEM((CG, H), jnp.float32),
            pltpu.VMEM((CROWS, H), jnp.float32),
            pltpu.VMEM((CROWS, H), jnp.float32),
            pltpu.SemaphoreType.DMA,
            pltpu.SemaphoreType.DMA,
        ],
    )
    def k(a_hbm, bv_hbm, ai_hbm, ei_hbm, out_hbm,
          ai_v, ei_v, ga0, gb0, ga1, gb1, oc0, oc1, sg0, sg1):
        wid = lax.axis_index("s") * nc + lax.axis_index("c")
        row_base = wid * RPW
        bufs = ((ga0, gb0, oc0, sg0), (ga1, gb1, oc1, sg1))

        pltpu.sync_copy(ai_hbm.at[pl.ds(row_base * MAX_NB, gpw)], ai_v)
        pltpu.sync_copy(ei_hbm.at[pl.ds(row_base * MAX_NB, gpw)], ei_v)

        def fire(c, b):
            ga, gb, _, sg = bufs[b]
            g0 = c * CG
            for j in range(CG // SUB):
                s = pl.ds(g0 + j * SUB, SUB)
                dsl = pl.ds(j * SUB, SUB)
                pltpu.async_copy(a_hbm.at[ai_v.at[s]], ga.at[dsl], sg)
                pltpu.async_copy(bv_hbm.at[ei_v.at[s]], gb.at[dsl], sg)

        def drain(b):
            ga, gb, _, sg = bufs[b]
            pltpu.make_async_copy(a_hbm.at[ai_v.at[pl.ds(0, CG)]], ga, sg).wait()
            pltpu.make_async_copy(bv_hbm.at[ei_v.at[pl.ds(0, CG)]], gb, sg).wait()

        def compute_store(c, b):
            ga, gb, oc, _ = bufs[b]

            def row(r, c2):
                rk = r * MAX_NB
                for h in range(H // 16):
                    sl = pl.ds(h * 16, 16)
                    acc = jnp.zeros((16,), jnp.float32)
                    for kk in range(MAX_NB):
                        v = ga[rk + kk, sl] + gb[rk + kk, sl]
                        acc = acc + jnp.maximum(v, 0.0)
                    oc[r, sl] = acc
                return c2

            lax.fori_loop(0, CROWS, row, 0)
            pltpu.sync_copy(oc, out_hbm.at[pl.ds(row_base + c * CROWS, CROWS)])

        fire(0, 0)

        def pair(i, carry):
            c0 = 2 * i
            fire(c0 + 1, 1)
            drain(0)
            compute_store(c0, 0)
            fire(c0 + 2, 0)
            drain(1)
            compute_store(c0 + 1, 1)
            return carry

        lax.fori_loop(0, NCH // 2 - 1, pair, 0)
        fire(NCH - 1, 1)
        drain(0)
        compute_store(NCH - 2, 0)
        drain(1)
        compute_store(NCH - 1, 1)

    return k(A, Bv, a_idx, e_idx)


def kernel(input_atom, input_bond, atom_graph, bond_graph, num_nbs,
           W_atom, W_U2, b_U2, W_U1, b_U1):
    w2a = W_U2[:H]
    w2b = W_U2[H:]
    w1a = W_U1[:H]
    w1b = W_U1[H:]
    b1 = b_U1.reshape(1, H)
    b2 = b_U2.reshape(1, H)

    a_core = atom_graph[..., 0] * NPAD + atom_graph[..., 1]
    valid = jnp.arange(MAX_NB, dtype=jnp.int32)[None, None, :] < num_nbs[:, :, None]
    boff_e = jnp.arange(B, dtype=jnp.int32)[:, None, None] * NEB
    spread = (jnp.arange(N * MAX_NB, dtype=jnp.int32) % NSENT).reshape(1, N, MAX_NB)
    e_core = jnp.where(valid, bond_graph[..., 0] * NEB + bond_graph[..., 1],
                       boff_e + N + spread)
    npr = NPAD - N
    boff_a = jnp.arange(B, dtype=jnp.int32)[:, None, None] * NPAD
    pidx = jnp.arange(npr * MAX_NB, dtype=jnp.int32).reshape(1, npr, MAX_NB)
    a_pad = jnp.broadcast_to(boff_a + pidx % N, (B, npr, MAX_NB))
    e_pad = jnp.broadcast_to(boff_e + N + pidx % NSENT, (B, npr, MAX_NB))
    a_flat = jnp.concatenate([a_core, a_pad], axis=1).reshape(-1)
    e_flat = jnp.concatenate([e_core, e_pad], axis=1).reshape(-1)

    bv = _tc_bond(input_bond, w2b)
    af, a_tab = _tc_init(input_atom, W_atom, w2a, b2)
    for d in range(DEPTH):
        nei = _sc_nei(a_tab, bv, a_flat, e_flat)
        if d < DEPTH - 1:
            af, a_tab = _tc_step(af, nei, w1a, w1b, b1, w2a, b2)
        else:
            out = _tc_final(af, nei, w1a, w1b, b1)
    return out


# speedup vs baseline: 1.0695x; 1.0602x over previous
import functools

import jax
import jax.numpy as jnp
from jax import lax
from jax.experimental import pallas as pl
from jax.experimental.pallas import tpu as pltpu
from jax.experimental.pallas import tpu_sc as plsc

B, N, E, MAX_NB, H, F_ATOM, F_BOND, DEPTH = 4, 12500, 12500, 10, 128, 89, 5, 3
NPAD = 12800
NP = B * NPAD
NEB = 13312
NPE = B * NEB
NSENT = NEB - N
NW = 32
CROWS = 16
SUB = 80
CG = CROWS * MAX_NB
RPW = NP // NW
NCH = RPW // CROWS
NEG = -1e30
TM = 512


def _bdot(a, b):
    return jnp.dot(a.astype(jnp.bfloat16), b.astype(jnp.bfloat16),
                   preferred_element_type=jnp.float32)


def _mm_init_body(x_ref, wat_ref, w2a_ref, b2_ref, af_ref, a_ref):
    af = _bdot(x_ref[0], wat_ref[...])
    af_ref[...] = af.astype(jnp.bfloat16)
    a_ref[...] = _bdot(af, w2a_ref[...]) + b2_ref[...]


def _mm_step_body(af_ref, nei_ref, w1a_ref, w1b_ref, b1_ref, w2a_ref, b2_ref,
                  af_out, a_out):
    x = _bdot(af_ref[...], w1a_ref[...]) + _bdot(nei_ref[...], w1b_ref[...])
    x = jnp.maximum(x + b1_ref[...], 0.0)
    af_out[...] = x.astype(jnp.bfloat16)
    a_out[...] = _bdot(x, w2a_ref[...]) + b2_ref[...]


def _mm_final_body(af_ref, nei_ref, w1a_ref, w1b_ref, b1_ref, out_ref):
    x = _bdot(af_ref[...], w1a_ref[...]) + _bdot(nei_ref[...], w1b_ref[...])
    out_ref[0] = jnp.maximum(x + b1_ref[...], 0.0)


def _mm_bond_body(bond_ref, w2b_ref, out_ref):
    v = _bdot(bond_ref[0], w2b_ref[...])
    row = pl.program_id(1) * TM + lax.broadcasted_iota(jnp.int32, (TM, H), 0)
    out_ref[...] = jnp.where(row < E, v, NEG)


def _full2(shape):
    n = len(shape)

    def im(*_):
        return (0,) * n

    return pl.BlockSpec(shape, im)


def _tc_init(x3, wat, w2a, b2):
    return pl.pallas_call(
        _mm_init_body,
        grid=(B, NPAD // TM),
        in_specs=[pl.BlockSpec((1, TM, F_ATOM), lambda b, j: (b, j, 0)),
                  _full2((F_ATOM, H)), _full2((H, H)), _full2((1, H))],
        out_specs=[pl.BlockSpec((TM, H), lambda b, j: (b * (NPAD // TM) + j, 0)),
                   pl.BlockSpec((TM, H), lambda b, j: (b * (NPAD // TM) + j, 0))],
        out_shape=[jax.ShapeDtypeStruct((NP, H), jnp.bfloat16),
                   jax.ShapeDtypeStruct((NP, H), jnp.float32)],
    )(x3, wat, w2a, b2)


def _tc_bond(bond3, w2b):
    return pl.pallas_call(
        _mm_bond_body,
        grid=(B, NEB // TM),
        in_specs=[pl.BlockSpec((1, TM, F_BOND),
                               lambda b, j: (b, jnp.minimum(j, NPAD // TM - 1), 0)),
                  _full2((F_BOND, H))],
        out_specs=pl.BlockSpec((TM, H), lambda b, j: (b * (NEB // TM) + j, 0)),
        out_shape=jax.ShapeDtypeStruct((NPE, H), jnp.float32),
    )(bond3, w2b)


def _mm_step_half_body(af_ref, nei_ref, w1a_ref, w1b_ref, b1_ref, w2a_ref,
                       b2_ref, afd_ref, ad_ref, af_out, a_out):
    del afd_ref, ad_ref
    _mm_step_body(af_ref, nei_ref, w1a_ref, w1b_ref, b1_ref, w2a_ref, b2_ref,
                  af_out, a_out)


def _tc_step_half(af, nei_h, w1a, w1b, b1, w2a, b2, af_dst, a_dst, half):
    ntiles = NP2 // TM
    af_rows = pl.BlockSpec((TM, H), lambda i: (half * ntiles + i, 0))
    rows = pl.BlockSpec((TM, H), lambda i: (i, 0))
    any_spec = pl.BlockSpec(memory_space=pl.ANY)
    return pl.pallas_call(
        _mm_step_half_body,
        grid=(ntiles,),
        in_specs=[af_rows, rows, _full2((H, H)), _full2((H, H)), _full2((1, H)),
                  _full2((H, H)), _full2((1, H)), any_spec, any_spec],
        out_specs=[af_rows, af_rows],
        out_shape=[jax.ShapeDtypeStruct((NP, H), jnp.bfloat16),
                   jax.ShapeDtypeStruct((NP, H), jnp.float32)],
        input_output_aliases={7: 0, 8: 1},
    )(af, nei_h, w1a, w1b, b1, w2a, b2, af_dst, a_dst)


def _tc_final_half(af, nei_h, w1a, w1b, b1, half):
    bh = B // NHALF
    af_rows = pl.BlockSpec(
        (TM, H), lambda b, j: ((half * bh + b) * (NPAD // TM) + j, 0))
    nei_rows = pl.BlockSpec((TM, H), lambda b, j: (b * (NPAD // TM) + j, 0))
    return pl.pallas_call(
        _mm_final_body,
        grid=(bh, NPAD // TM),
        in_specs=[af_rows, nei_rows, _full2((H, H)), _full2((H, H)),
                  _full2((1, H))],
        out_specs=pl.BlockSpec((1, TM, H), lambda b, j: (b, j, 0)),
        out_shape=jax.ShapeDtypeStruct((bh, N, H), jnp.float32),
    )(af, nei_h, w1a, w1b, b1)


NHALF = 2
NP2 = NP // NHALF
RPW2 = NP2 // NW
NCH2 = RPW2 // CROWS


def _sc_nei(A, Bv, a_idx, e_idx, half):
    info = plsc.get_sparse_core_info()
    nc = info.num_cores
    mesh = plsc.VectorSubcoreMesh(core_axis_name="c", subcore_axis_name="s")
    gpw = RPW2 * MAX_NB
    RPW_, NCH_ = RPW2, NCH2

    @functools.partial(
        pl.kernel,
        mesh=mesh,
        out_type=jax.ShapeDtypeStruct((NP2, H), jnp.float32),
        scratch_types=[
            pltpu.VMEM((gpw,), jnp.int32),
            pltpu.VMEM((gpw,), jnp.int32),
            pltpu.VMEM((CG, H), jnp.float32),
            pltpu.VMEM((CG, H), jnp.float32),
            pltpu.VMEM((CG, H), jnp.float32),
            pltpu.VMEM((CG, H), jnp.float32),
            pltpu.VMEM((CROWS, H), jnp.float32),
            pltpu.VMEM((CROWS, H), jnp.float32),
            pltpu.SemaphoreType.DMA,
            pltpu.SemaphoreType.DMA,
        ],
    )
    def k(a_hbm, bv_hbm, ai_hbm, ei_hbm, out_hbm,
          ai_v, ei_v, ga0, gb0, ga1, gb1, oc0, oc1, sg0, sg1):
        wid = lax.axis_index("s") * nc + lax.axis_index("c")
        row_base = wid * RPW_
        bufs = ((ga0, gb0, oc0, sg0), (ga1, gb1, oc1, sg1))

        slot0 = (half * NP2 + row_base) * MAX_NB
        pltpu.sync_copy(ai_hbm.at[pl.ds(slot0, gpw)], ai_v)
        pltpu.sync_copy(ei_hbm.at[pl.ds(slot0, gpw)], ei_v)

        def fire(c, b):
            ga, gb, _, sg = bufs[b]
            g0 = c * CG
            for j in range(CG // SUB):
                s = pl.ds(g0 + j * SUB, SUB)
                dsl = pl.ds(j * SUB, SUB)
                pltpu.async_copy(a_hbm.at[ai_v.at[s]], ga.at[dsl], sg)
                pltpu.async_copy(bv_hbm.at[ei_v.at[s]], gb.at[dsl], sg)

        def drain(b):
            ga, gb, _, sg = bufs[b]
            pltpu.make_async_copy(a_hbm.at[ai_v.at[pl.ds(0, CG)]], ga, sg).wait()
            pltpu.make_async_copy(bv_hbm.at[ei_v.at[pl.ds(0, CG)]], gb, sg).wait()

        def compute_store(c, b):
            ga, gb, oc, _ = bufs[b]

            def row(r, c2):
                rk = r * MAX_NB
                for h in range(H // 16):
                    sl = pl.ds(h * 16, 16)
                    acc = jnp.zeros((16,), jnp.float32)
                    for kk in range(MAX_NB):
                        v = ga[rk + kk, sl] + gb[rk + kk, sl]
                        acc = acc + jnp.maximum(v, 0.0)
                    oc[r, sl] = acc
                return c2

            lax.fori_loop(0, CROWS, row, 0)
            pltpu.sync_copy(oc, out_hbm.at[pl.ds(row_base + c * CROWS, CROWS)])

        fire(0, 0)

        def pair(i, carry):
            c0 = 2 * i
            fire(c0 + 1, 1)
            drain(0)
            compute_store(c0, 0)
            fire(c0 + 2, 0)
            drain(1)
            compute_store(c0 + 1, 1)
            return carry

        lax.fori_loop(0, NCH_ // 2 - 1, pair, 0)
        fire(NCH_ - 1, 1)
        drain(0)
        compute_store(NCH_ - 2, 0)
        drain(1)
        compute_store(NCH_ - 1, 1)

    return k(A, Bv, a_idx, e_idx)


def kernel(input_atom, input_bond, atom_graph, bond_graph, num_nbs,
           W_atom, W_U2, b_U2, W_U1, b_U1):
    w2a = W_U2[:H]
    w2b = W_U2[H:]
    w1a = W_U1[:H]
    w1b = W_U1[H:]
    b1 = b_U1.reshape(1, H)
    b2 = b_U2.reshape(1, H)

    a_core = atom_graph[..., 0] * NPAD + atom_graph[..., 1]
    valid = jnp.arange(MAX_NB, dtype=jnp.int32)[None, None, :] < num_nbs[:, :, None]
    boff_e = jnp.arange(B, dtype=jnp.int32)[:, None, None] * NEB
    spread = (jnp.arange(N * MAX_NB, dtype=jnp.int32) % NSENT).reshape(1, N, MAX_NB)
    e_core = jnp.where(valid, bond_graph[..., 0] * NEB + bond_graph[..., 1],
                       boff_e + N + spread)
    npr = NPAD - N
    boff_a = jnp.arange(B, dtype=jnp.int32)[:, None, None] * NPAD
    pidx = jnp.arange(npr * MAX_NB, dtype=jnp.int32).reshape(1, npr, MAX_NB)
    a_pad = jnp.broadcast_to(boff_a + pidx % N, (B, npr, MAX_NB))
    e_pad = jnp.broadcast_to(boff_e + N + pidx % NSENT, (B, npr, MAX_NB))
    a_flat = jnp.concatenate([a_core, a_pad], axis=1).reshape(-1)
    e_flat = jnp.concatenate([e_core, e_pad], axis=1).reshape(-1)

    bv = _tc_bond(input_bond, w2b)
    af, a_tab = _tc_init(input_atom, W_atom, w2a, b2)
    af_dst = jnp.zeros((NP, H), jnp.bfloat16)
    a_dst = jnp.zeros((NP, H), jnp.float32)
    for d in range(DEPTH):
        nei0 = _sc_nei(a_tab, bv, a_flat, e_flat, 0)
        nei1 = _sc_nei(a_tab, bv, a_flat, e_flat, 1)
        if d < DEPTH - 1:
            afp, ap = _tc_step_half(af, nei0, w1a, w1b, b1, w2a, b2,
                                    af_dst, a_dst, 0)
            af_new, a_new = _tc_step_half(af, nei1, w1a, w1b, b1, w2a, b2,
                                          afp, ap, 1)
            af_dst, a_dst = af, a_tab
            af, a_tab = af_new, a_new
        else:
            out0 = _tc_final_half(af, nei0, w1a, w1b, b1, 0)
            out1 = _tc_final_half(af, nei1, w1a, w1b, b1, 1)
            out = jnp.concatenate([out0, out1], axis=0)
    return out


# speedup vs baseline: 1.0696x; 1.0001x over previous
import functools

import jax
import jax.numpy as jnp
from jax import lax
from jax.experimental import pallas as pl
from jax.experimental.pallas import tpu as pltpu
from jax.experimental.pallas import tpu_sc as plsc

B, N, E, MAX_NB, H, F_ATOM, F_BOND, DEPTH = 4, 12500, 12500, 10, 128, 89, 5, 3
NPAD = 12800
NP = B * NPAD
NEB = 13312
NPE = B * NEB
NSENT = NEB - N
NW = 32
CROWS = 16
SUB = 160
CG = CROWS * MAX_NB
RPW = NP // NW
NCH = RPW // CROWS
NEG = -1e30
TM = 512


def _bdot(a, b):
    return jnp.dot(a.astype(jnp.bfloat16), b.astype(jnp.bfloat16),
                   preferred_element_type=jnp.float32)


def _mm_init_body(x_ref, wat_ref, w2a_ref, b2_ref, af_ref, a_ref):
    af = _bdot(x_ref[0], wat_ref[...])
    af_ref[...] = af.astype(jnp.bfloat16)
    a_ref[...] = _bdot(af, w2a_ref[...]) + b2_ref[...]


def _mm_step_body(af_ref, nei_ref, w1a_ref, w1b_ref, b1_ref, w2a_ref, b2_ref,
                  af_out, a_out):
    x = _bdot(af_ref[...], w1a_ref[...]) + _bdot(nei_ref[...], w1b_ref[...])
    x = jnp.maximum(x + b1_ref[...], 0.0)
    af_out[...] = x.astype(jnp.bfloat16)
    a_out[...] = _bdot(x, w2a_ref[...]) + b2_ref[...]


def _mm_final_body(af_ref, nei_ref, w1a_ref, w1b_ref, b1_ref, out_ref):
    x = _bdot(af_ref[...], w1a_ref[...]) + _bdot(nei_ref[...], w1b_ref[...])
    out_ref[0] = jnp.maximum(x + b1_ref[...], 0.0)


def _mm_bond_body(bond_ref, w2b_ref, out_ref):
    v = _bdot(bond_ref[0], w2b_ref[...])
    row = pl.program_id(1) * TM + lax.broadcasted_iota(jnp.int32, (TM, H), 0)
    out_ref[...] = jnp.where(row < E, v, NEG)


def _full2(shape):
    n = len(shape)

    def im(*_):
        return (0,) * n

    return pl.BlockSpec(shape, im)


def _tc_init(x3, wat, w2a, b2):
    return pl.pallas_call(
        _mm_init_body,
        grid=(B, NPAD // TM),
        in_specs=[pl.BlockSpec((1, TM, F_ATOM), lambda b, j: (b, j, 0)),
                  _full2((F_ATOM, H)), _full2((H, H)), _full2((1, H))],
        out_specs=[pl.BlockSpec((TM, H), lambda b, j: (b * (NPAD // TM) + j, 0)),
                   pl.BlockSpec((TM, H), lambda b, j: (b * (NPAD // TM) + j, 0))],
        out_shape=[jax.ShapeDtypeStruct((NP, H), jnp.bfloat16),
                   jax.ShapeDtypeStruct((NP, H), jnp.float32)],
    )(x3, wat, w2a, b2)


def _tc_bond(bond3, w2b):
    return pl.pallas_call(
        _mm_bond_body,
        grid=(B, NEB // TM),
        in_specs=[pl.BlockSpec((1, TM, F_BOND),
                               lambda b, j: (b, jnp.minimum(j, NPAD // TM - 1), 0)),
                  _full2((F_BOND, H))],
        out_specs=pl.BlockSpec((TM, H), lambda b, j: (b * (NEB // TM) + j, 0)),
        out_shape=jax.ShapeDtypeStruct((NPE, H), jnp.float32),
    )(bond3, w2b)


def _mm_step_half_body(af_ref, nei_ref, w1a_ref, w1b_ref, b1_ref, w2a_ref,
                       b2_ref, afd_ref, ad_ref, af_out, a_out):
    del afd_ref, ad_ref
    _mm_step_body(af_ref, nei_ref, w1a_ref, w1b_ref, b1_ref, w2a_ref, b2_ref,
                  af_out, a_out)


def _tc_step_half(af, nei_h, w1a, w1b, b1, w2a, b2, af_dst, a_dst, half):
    ntiles = NP2 // TM
    af_rows = pl.BlockSpec((TM, H), lambda i: (half * ntiles + i, 0))
    rows = pl.BlockSpec((TM, H), lambda i: (i, 0))
    any_spec = pl.BlockSpec(memory_space=pl.ANY)
    return pl.pallas_call(
        _mm_step_half_body,
        grid=(ntiles,),
        in_specs=[af_rows, rows, _full2((H, H)), _full2((H, H)), _full2((1, H)),
                  _full2((H, H)), _full2((1, H)), any_spec, any_spec],
        out_specs=[af_rows, af_rows],
        out_shape=[jax.ShapeDtypeStruct((NP, H), jnp.bfloat16),
                   jax.ShapeDtypeStruct((NP, H), jnp.float32)],
        input_output_aliases={7: 0, 8: 1},
    )(af, nei_h, w1a, w1b, b1, w2a, b2, af_dst, a_dst)


def _tc_final_half(af, nei_h, w1a, w1b, b1, half):
    bh = B // NHALF
    af_rows = pl.BlockSpec(
        (TM, H), lambda b, j: ((half * bh + b) * (NPAD // TM) + j, 0))
    nei_rows = pl.BlockSpec((TM, H), lambda b, j: (b * (NPAD // TM) + j, 0))
    return pl.pallas_call(
        _mm_final_body,
        grid=(bh, NPAD // TM),
        in_specs=[af_rows, nei_rows, _full2((H, H)), _full2((H, H)),
                  _full2((1, H))],
        out_specs=pl.BlockSpec((1, TM, H), lambda b, j: (b, j, 0)),
        out_shape=jax.ShapeDtypeStruct((bh, N, H), jnp.float32),
    )(af, nei_h, w1a, w1b, b1)


NHALF = 2
NP2 = NP // NHALF
RPW2 = NP2 // NW
NCH2 = RPW2 // CROWS


def _sc_nei(A, Bv, a_idx, e_idx, half):
    info = plsc.get_sparse_core_info()
    nc = info.num_cores
    mesh = plsc.VectorSubcoreMesh(core_axis_name="c", subcore_axis_name="s")
    gpw = RPW2 * MAX_NB
    RPW_, NCH_ = RPW2, NCH2

    @functools.partial(
        pl.kernel,
        mesh=mesh,
        out_type=jax.ShapeDtypeStruct((NP2, H), jnp.float32),
        scratch_types=[
            pltpu.VMEM((gpw,), jnp.int32),
            pltpu.VMEM((gpw,), jnp.int32),
            pltpu.VMEM((CG, H), jnp.float32),
            pltpu.VMEM((CG, H), jnp.float32),
            pltpu.VMEM((CG, H), jnp.float32),
            pltpu.VMEM((CG, H), jnp.float32),
            pltpu.VMEM((CROWS, H), jnp.float32),
            pltpu.VMEM((CROWS, H), jnp.float32),
            pltpu.SemaphoreType.DMA,
            pltpu.SemaphoreType.DMA,
        ],
    )
    def k(a_hbm, bv_hbm, ai_hbm, ei_hbm, out_hbm,
          ai_v, ei_v, ga0, gb0, ga1, gb1, oc0, oc1, sg0, sg1):
        wid = lax.axis_index("s") * nc + lax.axis_index("c")
        row_base = wid * RPW_
        bufs = ((ga0, gb0, oc0, sg0), (ga1, gb1, oc1, sg1))

        slot0 = (half * NP2 + row_base) * MAX_NB
        pltpu.sync_copy(ai_hbm.at[pl.ds(slot0, gpw)], ai_v)
        pltpu.sync_copy(ei_hbm.at[pl.ds(slot0, gpw)], ei_v)

        def fire(c, b):
            ga, gb, _, sg = bufs[b]
            g0 = c * CG
            for j in range(CG // SUB):
                s = pl.ds(g0 + j * SUB, SUB)
                dsl = pl.ds(j * SUB, SUB)
                pltpu.async_copy(a_hbm.at[ai_v.at[s]], ga.at[dsl], sg)
                pltpu.async_copy(bv_hbm.at[ei_v.at[s]], gb.at[dsl], sg)

        def drain(b):
            ga, gb, _, sg = bufs[b]
            pltpu.make_async_copy(a_hbm.at[ai_v.at[pl.ds(0, CG)]], ga, sg).wait()
            pltpu.make_async_copy(bv_hbm.at[ei_v.at[pl.ds(0, CG)]], gb, sg).wait()

        def compute_store(c, b):
            ga, gb, oc, _ = bufs[b]

            def row(r, c2):
                rk = r * MAX_NB
                for h in range(H // 16):
                    sl = pl.ds(h * 16, 16)
                    acc = jnp.zeros((16,), jnp.float32)
                    for kk in range(MAX_NB):
                        v = ga[rk + kk, sl] + gb[rk + kk, sl]
                        acc = acc + jnp.maximum(v, 0.0)
                    oc[r, sl] = acc
                return c2

            lax.fori_loop(0, CROWS, row, 0)
            pltpu.sync_copy(oc, out_hbm.at[pl.ds(row_base + c * CROWS, CROWS)])

        fire(0, 0)

        def pair(i, carry):
            c0 = 2 * i
            fire(c0 + 1, 1)
            drain(0)
            compute_store(c0, 0)
            fire(c0 + 2, 0)
            drain(1)
            compute_store(c0 + 1, 1)
            return carry

        lax.fori_loop(0, NCH_ // 2 - 1, pair, 0)
        fire(NCH_ - 1, 1)
        drain(0)
        compute_store(NCH_ - 2, 0)
        drain(1)
        compute_store(NCH_ - 1, 1)

    return k(A, Bv, a_idx, e_idx)


def kernel(input_atom, input_bond, atom_graph, bond_graph, num_nbs,
           W_atom, W_U2, b_U2, W_U1, b_U1):
    w2a = W_U2[:H]
    w2b = W_U2[H:]
    w1a = W_U1[:H]
    w1b = W_U1[H:]
    b1 = b_U1.reshape(1, H)
    b2 = b_U2.reshape(1, H)

    a_core = atom_graph[..., 0] * NPAD + atom_graph[..., 1]
    valid = jnp.arange(MAX_NB, dtype=jnp.int32)[None, None, :] < num_nbs[:, :, None]
    boff_e = jnp.arange(B, dtype=jnp.int32)[:, None, None] * NEB
    spread = (jnp.arange(N * MAX_NB, dtype=jnp.int32) % NSENT).reshape(1, N, MAX_NB)
    e_core = jnp.where(valid, bond_graph[..., 0] * NEB + bond_graph[..., 1],
                       boff_e + N + spread)
    npr = NPAD - N
    boff_a = jnp.arange(B, dtype=jnp.int32)[:, None, None] * NPAD
    pidx = jnp.arange(npr * MAX_NB, dtype=jnp.int32).reshape(1, npr, MAX_NB)
    a_pad = jnp.broadcast_to(boff_a + pidx % N, (B, npr, MAX_NB))
    e_pad = jnp.broadcast_to(boff_e + N + pidx % NSENT, (B, npr, MAX_NB))
    a_flat = jnp.concatenate([a_core, a_pad], axis=1).reshape(-1)
    e_flat = jnp.concatenate([e_core, e_pad], axis=1).reshape(-1)

    bv = _tc_bond(input_bond, w2b)
    af, a_tab = _tc_init(input_atom, W_atom, w2a, b2)
    af_dst = jnp.zeros((NP, H), jnp.bfloat16)
    a_dst = jnp.zeros((NP, H), jnp.float32)
    for d in range(DEPTH):
        nei0 = _sc_nei(a_tab, bv, a_flat, e_flat, 0)
        nei1 = _sc_nei(a_tab, bv, a_flat, e_flat, 1)
        if d < DEPTH - 1:
            afp, ap = _tc_step_half(af, nei0, w1a, w1b, b1, w2a, b2,
                                    af_dst, a_dst, 0)
            af_new, a_new = _tc_step_half(af, nei1, w1a, w1b, b1, w2a, b2,
                                          afp, ap, 1)
            af_dst, a_dst = af, a_tab
            af, a_tab = af_new, a_new
        else:
            out0 = _tc_final_half(af, nei0, w1a, w1b, b1, 0)
            out1 = _tc_final_half(af, nei1, w1a, w1b, b1, 1)
            out = jnp.concatenate([out0, out1], axis=0)
    return out
